# Initial kernel scaffold; baseline (speedup 1.0000x reference)
import jax
import jax.numpy as jnp
from jax.experimental import pallas as pl


def kernel(target, model_points, idx, H):
    raise NotImplementedError("write your pallas kernel here")


# speedup vs baseline: 3.1288x; 3.1288x over previous
import functools

import jax
import jax.numpy as jnp
from jax.experimental import pallas as pl

_SYM = (0, 3, 5, 8)
_N = 1024
_CH = 256


def _body(t_col_ref, m_row_ref, t_row_ref, h_ref, omin_ref, opair_ref):
    mx = m_row_ref[0, 0:1, :]
    my = m_row_ref[0, 1:2, :]
    mz = m_row_ref[0, 2:3, :]
    px = h_ref[0, 0, 0] * mx + h_ref[0, 0, 1] * my + h_ref[0, 0, 2] * mz + h_ref[0, 0, 3]
    py = h_ref[0, 1, 0] * mx + h_ref[0, 1, 1] * my + h_ref[0, 1, 2] * mz + h_ref[0, 1, 3]
    pz = h_ref[0, 2, 0] * mx + h_ref[0, 2, 1] * my + h_ref[0, 2, 2] * mz + h_ref[0, 2, 3]

    tx_r = t_row_ref[0, 0:1, :]
    ty_r = t_row_ref[0, 1:2, :]
    tz_r = t_row_ref[0, 2:3, :]
    opair_ref[0] = jnp.sqrt((px - tx_r) ** 2 + (py - ty_r) ** 2 + (pz - tz_r) ** 2)

    macc = jnp.full((1, _N), jnp.inf, dtype=jnp.float32)
    for c in range(_N // _CH):
        txc = t_col_ref[0, c * _CH:(c + 1) * _CH, 0:1]
        tyc = t_col_ref[0, c * _CH:(c + 1) * _CH, 1:2]
        tzc = t_col_ref[0, c * _CH:(c + 1) * _CH, 2:3]
        d2 = (txc - px) ** 2 + (tyc - py) ** 2 + (tzc - pz) ** 2
        macc = jnp.minimum(macc, jnp.min(d2, axis=0, keepdims=True))
    omin_ref[0] = jnp.sqrt(macc)


@jax.jit
def kernel(target, model_points, idx, H):
    bs, n, _ = target.shape
    m_rows = model_points.transpose(0, 2, 1)
    t_rows = target.transpose(0, 2, 1)

    omin, opair = pl.pallas_call(
        _body,
        grid=(bs,),
        in_specs=[
            pl.BlockSpec((1, n, 3), lambda b: (b, 0, 0)),
            pl.BlockSpec((1, 3, n), lambda b: (b, 0, 0)),
            pl.BlockSpec((1, 3, n), lambda b: (b, 0, 0)),
            pl.BlockSpec((1, 4, 4), lambda b: (b, 0, 0)),
        ],
        out_specs=[
            pl.BlockSpec((1, 1, n), lambda b: (b, 0, 0)),
            pl.BlockSpec((1, 1, n), lambda b: (b, 0, 0)),
        ],
        out_shape=[
            jax.ShapeDtypeStruct((bs, 1, n), jnp.float32),
            jax.ShapeDtypeStruct((bs, 1, n), jnp.float32),
        ],
    )(target, m_rows, t_rows, H)

    omin = omin.reshape(bs, n)
    opair = opair.reshape(bs, n)
    is_sym = jnp.isin(idx[:, 0], jnp.array(_SYM, dtype=jnp.int32))
    return jnp.where(is_sym, omin.mean(axis=1), opair.mean(axis=1))
